# baseline (speedup 1.0000x reference)
import functools

import jax
import jax.numpy as jnp
from jax import lax
from jax.experimental import pallas as pl
from jax.experimental.pallas import tpu as pltpu
from jax.experimental.pallas import tpu_sc as plsc

_SUPPORT = 30
_EPS = 0.001
_B = 131072
_N = 2 * _SUPPORT + 1

_NC = 2
_NS = 16
_NW = _NC * _NS
_ROWS_PER_W = _B // _NW
_CHUNK = 1024
_NCHUNK = _ROWS_PER_W // _CHUNK

_LN2 = 0.6931471805599453
_SQRT2 = 1.4142135623730951


def _newton_sqrt(a):
    i = lax.bitcast_convert_type(a, jnp.int32)
    i = 0x5F3759DF - (i >> 1)
    r = lax.bitcast_convert_type(i, jnp.float32)
    for _ in range(3):
        r = r * (1.5 - 0.5 * a * r * r)
    return a * r


def _log_f32(s):
    bits = lax.bitcast_convert_type(s, jnp.int32)
    e = (bits >> 23) - 127
    m = lax.bitcast_convert_type((bits & 0x007FFFFF) | 0x3F800000, jnp.float32)
    big = m > _SQRT2
    m = jnp.where(big, m * 0.5, m)
    e = (e + big.astype(jnp.int32)).astype(jnp.float32)
    t = (m - 1.0) / (m + 1.0)
    z = t * t
    poly = 2.0 + z * (2.0 / 3.0 + z * (2.0 / 5.0 + z * (2.0 / 7.0 + z * (2.0 / 9.0))))
    return e * _LN2 + t * poly


def _twohot_params(t):
    xs = jnp.sign(t) * (_newton_sqrt(jnp.abs(t) + 1.0) - 1.0) + _EPS * t
    xs = jnp.clip(xs, -float(_SUPPORT), float(_SUPPORT))
    tr = xs.astype(jnp.int32)
    fl = tr - (xs < tr.astype(jnp.float32)).astype(jnp.int32)
    under = xs - fl.astype(jnp.float32)
    fp = 1.0 - under
    fi = fl + _SUPPORT
    ui = fi + 1
    mask = ui > 2 * _SUPPORT
    up = jnp.where(mask, 0.0, under)
    ui = jnp.where(mask, 0, ui)
    return fi, ui, fp, up


def _sc_body(x_hbm, t_hbm, out_hbm, buf0, buf1, tbuf0, tbuf1, res_v, sem0, sem1,
             *, rows_per_w, nchunk, col_len):
    wid = lax.axis_index("s") * _NC + lax.axis_index("c")
    row0 = wid * rows_per_w

    lane = lax.iota(jnp.int32, 16)
    inv_b = 1.0 / _B

    bufs = (buf0, buf1)
    sems = (sem0, sem1)
    copies = [None, None]

    tbufs = (tbuf0, tbuf1)

    def start(i):
        base = row0 + i * _CHUNK
        copies[i % 2] = [
            pltpu.async_copy(
                x_hbm.at[pl.ds(c * col_len + base, _CHUNK)],
                bufs[i % 2].at[pl.ds(c * _CHUNK, _CHUNK)],
                sems[i % 2],
            )
            for c in range(_N)
        ] + [
            pltpu.async_copy(
                t_hbm.at[pl.ds(base, _CHUNK)], tbufs[i % 2], sems[i % 2]
            )
        ]

    start(0)
    total = jnp.zeros((16,), jnp.float32)
    for i in range(nchunk):
        if i + 1 < nchunk:
            start(i + 1)
        for cp in copies[i % 2]:
            cp.wait()
        cur = bufs[i % 2]
        tcur = tbufs[i % 2]

        def group(g, acc, cur=cur, tcur=tcur):
            base = g * 16 + lane
            g16 = g * 16
            t = tcur[pl.ds(g16, 16)]
            fi, ui, fp, up = _twohot_params(t)
            ss = [jnp.zeros((16,), jnp.float32) for _ in range(4)]
            for c in range(_N):
                ss[c % 4] = ss[c % 4] + jnp.exp(cur[pl.ds(g16 + c * _CHUNK, 16)])
            s = (ss[0] + ss[1]) + (ss[2] + ss[3])
            lse = _log_f32(s)
            vf = plsc.load_gather(cur, [base + (fi << 10)])
            vu = plsc.load_gather(cur, [base + (ui << 10)])
            return acc + (lse - fp * vf - up * vu)

        total = lax.fori_loop(0, _CHUNK // 16, group, total)

    res_v[...] = total * inv_b
    pltpu.sync_copy(res_v, out_hbm.at[wid])


def _sc_call(x, t, rows):
    rows_per_w = rows // _NW
    nchunk = rows_per_w // _CHUNK
    mesh = plsc.VectorSubcoreMesh(core_axis_name="c", subcore_axis_name="s")
    k = functools.partial(
        pl.kernel,
        mesh=mesh,
        compiler_params=pltpu.CompilerParams(
            needs_layout_passes=False, use_tc_tiling_on_sc=False
        ),
        out_type=jax.ShapeDtypeStruct((_NW, 16), jnp.float32),
        scratch_types=[
            pltpu.VMEM((_CHUNK * _N,), jnp.float32),
            pltpu.VMEM((_CHUNK * _N,), jnp.float32),
            pltpu.VMEM((_CHUNK,), jnp.float32),
            pltpu.VMEM((_CHUNK,), jnp.float32),
            pltpu.VMEM((16,), jnp.float32),
            pltpu.SemaphoreType.DMA,
            pltpu.SemaphoreType.DMA,
        ],
    )(functools.partial(_sc_body, rows_per_w=rows_per_w, nchunk=nchunk, col_len=rows))
    return k(x, t)


def kernel(pred_logits, target):
    half = _B // 2
    tflat = target.reshape(-1)
    acc = None
    for h in range(2):
        xh = jnp.transpose(pred_logits)[:, h * half:(h + 1) * half].reshape(-1)
        o = _sc_call(xh, tflat[h * half:(h + 1) * half], half)
        acc = o if acc is None else acc + o
    return jnp.sum(acc)


# speedup vs baseline: 1.3730x; 1.3730x over previous
import functools

import jax
import jax.numpy as jnp
from jax import lax
from jax.experimental import pallas as pl
from jax.experimental.pallas import tpu as pltpu
from jax.experimental.pallas import tpu_sc as plsc

_SUPPORT = 30
_EPS = 0.001
_B = 131072
_N = 2 * _SUPPORT + 1

_NC = 2
_NS = 16
_NW = _NC * _NS
_ROWS_PER_W = _B // _NW
_CHUNK = 1024
_NCHUNK = _ROWS_PER_W // _CHUNK

_LN2 = 0.6931471805599453
_SQRT2 = 1.4142135623730951


def _newton_sqrt(a):
    i = lax.bitcast_convert_type(a, jnp.int32)
    i = 0x5F3759DF - (i >> 1)
    r = lax.bitcast_convert_type(i, jnp.float32)
    for _ in range(3):
        r = r * (1.5 - 0.5 * a * r * r)
    return a * r


def _log_f32(s):
    bits = lax.bitcast_convert_type(s, jnp.int32)
    e = (bits >> 23) - 127
    m = lax.bitcast_convert_type((bits & 0x007FFFFF) | 0x3F800000, jnp.float32)
    big = m > _SQRT2
    m = jnp.where(big, m * 0.5, m)
    e = (e + big.astype(jnp.int32)).astype(jnp.float32)
    t = (m - 1.0) / (m + 1.0)
    z = t * t
    poly = 2.0 + z * (2.0 / 3.0 + z * (2.0 / 5.0 + z * (2.0 / 7.0 + z * (2.0 / 9.0))))
    return e * _LN2 + t * poly


def _twohot_params(t):
    xs = jnp.sign(t) * (_newton_sqrt(jnp.abs(t) + 1.0) - 1.0) + _EPS * t
    xs = jnp.clip(xs, -float(_SUPPORT), float(_SUPPORT))
    tr = xs.astype(jnp.int32)
    fl = tr - (xs < tr.astype(jnp.float32)).astype(jnp.int32)
    under = xs - fl.astype(jnp.float32)
    fp = 1.0 - under
    fi = fl + _SUPPORT
    ui = fi + 1
    mask = ui > 2 * _SUPPORT
    up = jnp.where(mask, 0.0, under)
    ui = jnp.where(mask, 0, ui)
    return fi, ui, fp, up


def _sc_body(x_hbm, t_hbm, out_hbm, buf0, buf1, tbuf0, tbuf1, res_v, sem0, sem1,
             *, rows_per_w, nchunk, col_len):
    wid = lax.axis_index("s") * _NC + lax.axis_index("c")
    row0 = wid * rows_per_w

    lane = lax.iota(jnp.int32, 16)
    inv_b = 1.0 / _B

    bufs = (buf0, buf1)
    sems = (sem0, sem1)
    copies = [None, None]

    tbufs = (tbuf0, tbuf1)

    def start(i):
        base = row0 + i * _CHUNK
        copies[i % 2] = [
            pltpu.async_copy(
                x_hbm.at[pl.ds(c * col_len + base, _CHUNK)],
                bufs[i % 2].at[pl.ds(c * _CHUNK, _CHUNK)],
                sems[i % 2],
            )
            for c in range(_N)
        ] + [
            pltpu.async_copy(
                t_hbm.at[pl.ds(base, _CHUNK)], tbufs[i % 2], sems[i % 2]
            )
        ]

    start(0)
    total = jnp.zeros((16,), jnp.float32)
    for i in range(nchunk):
        if i + 1 < nchunk:
            start(i + 1)
        for cp in copies[i % 2]:
            cp.wait()
        cur = bufs[i % 2]
        tcur = tbufs[i % 2]

        def group(g, acc, cur=cur, tcur=tcur):
            base = g * 16 + lane
            g16 = g * 16
            t = tcur[pl.ds(g16, 16)]
            fi, ui, fp, up = _twohot_params(t)
            ss = [jnp.zeros((16,), jnp.float32) for _ in range(4)]
            for c in range(_N):
                ss[c % 4] = ss[c % 4] + jnp.exp(cur[pl.ds(g16 + c * _CHUNK, 16)])
            s = (ss[0] + ss[1]) + (ss[2] + ss[3])
            lse = _log_f32(s)
            vf = plsc.load_gather(cur, [base + (fi << 10)])
            vu = plsc.load_gather(cur, [base + (ui << 10)])
            return acc + (lse - fp * vf - up * vu)

        total = lax.fori_loop(0, _CHUNK // 16, group, total)

    res_v[...] = total * inv_b
    pltpu.sync_copy(res_v, out_hbm.at[wid])


def _sc_call(x, t, rows):
    rows_per_w = rows // _NW
    nchunk = rows_per_w // _CHUNK
    mesh = plsc.VectorSubcoreMesh(core_axis_name="c", subcore_axis_name="s")
    k = functools.partial(
        pl.kernel,
        mesh=mesh,
        compiler_params=pltpu.CompilerParams(
            needs_layout_passes=False, use_tc_tiling_on_sc=False
        ),
        out_type=jax.ShapeDtypeStruct((_NW, 16), jnp.float32),
        scratch_types=[
            pltpu.VMEM((_CHUNK * _N,), jnp.float32),
            pltpu.VMEM((_CHUNK * _N,), jnp.float32),
            pltpu.VMEM((_CHUNK,), jnp.float32),
            pltpu.VMEM((_CHUNK,), jnp.float32),
            pltpu.VMEM((16,), jnp.float32),
            pltpu.SemaphoreType.DMA,
            pltpu.SemaphoreType.DMA,
        ],
    )(functools.partial(_sc_body, rows_per_w=rows_per_w, nchunk=nchunk, col_len=rows))
    return k(x, t)


def kernel(pred_logits, target):
    xflat = jnp.transpose(pred_logits).reshape(-1)
    out = _sc_call(xflat, target.reshape(-1), _B)
    return jnp.sum(out)
